# baseline (speedup 1.0000x reference)
import functools

import jax
import jax.numpy as jnp
from jax import lax
from jax.experimental import pallas as pl
from jax.experimental.pallas import tpu as pltpu
from jax.experimental.pallas import tpu_sc as plsc

L = 16


@functools.partial(jax.jit, static_argnums=(6, 7))
def _sc_run(heads, rels, tails, user, ent_table, rel_table, B, D):
    info = plsc.get_sparse_core_info()
    NC, NS = info.num_cores, info.num_subcores
    NW = NC * NS
    BPW = B // NW
    C = 128
    NCH = BPW // C
    assert BPW * NW == B and NCH * C == BPW

    mesh = plsc.VectorSubcoreMesh(core_axis_name="c", subcore_axis_name="s")

    @functools.partial(
        pl.kernel,
        mesh=mesh,
        compiler_params=pltpu.CompilerParams(
            needs_layout_passes=False, use_tc_tiling_on_sc=False),
        out_type=(
            jax.ShapeDtypeStruct((B,), jnp.float32),
            jax.ShapeDtypeStruct((NW, L), jnp.float32),
        ),
        scratch_types=[
            pltpu.VMEM((C,), jnp.int32),
            pltpu.VMEM((C,), jnp.int32),
            pltpu.VMEM((C,), jnp.int32),
            pltpu.VMEM((C, D), jnp.float32),
            pltpu.VMEM((C, D), jnp.float32),
            pltpu.VMEM((C, D), jnp.float32),
            pltpu.VMEM((C, D), jnp.float32),
            pltpu.VMEM((BPW,), jnp.float32),
            pltpu.VMEM((L,), jnp.float32),
            pltpu.SemaphoreType.DMA,
        ],
    )
    def k(heads_hbm, rels_hbm, tails_hbm, user_hbm, ent_hbm, rel_hbm,
          score_out, norm_out,
          hidx, ridx, tidx, hbuf, rbuf, tbuf, ubuf, sbuf, nbuf, sem):
        wid = lax.axis_index("s") * NC + lax.axis_index("c")
        base = wid * BPW
        lane = lax.iota(jnp.int32, L)
        norm_vec = jnp.zeros((L,), jnp.float32)

        for c in range(NCH):
            r0 = base + c * C
            pltpu.sync_copy(heads_hbm.at[pl.ds(r0, C)], hidx)
            pltpu.sync_copy(rels_hbm.at[pl.ds(r0, C)], ridx)
            pltpu.sync_copy(tails_hbm.at[pl.ds(r0, C)], tidx)
            cp_h = pltpu.async_copy(ent_hbm.at[hidx], hbuf, sem)
            cp_r = pltpu.async_copy(rel_hbm.at[ridx], rbuf, sem)
            cp_t = pltpu.async_copy(ent_hbm.at[tidx], tbuf, sem)
            cp_u = pltpu.async_copy(user_hbm.at[pl.ds(r0, C)], ubuf, sem)
            cp_h.wait()
            cp_r.wait()
            cp_t.wait()
            cp_u.wait()

            def group_body(g, nv, _c=c):
                rowv = lane + g * L
                acc = jnp.zeros((L,), jnp.float32)
                nacc = jnp.zeros((L,), jnp.float32)
                for d in range(D):
                    colv = jnp.full((L,), d, jnp.int32)
                    h = plsc.load_gather(hbuf, [rowv, colv])
                    r = plsc.load_gather(rbuf, [rowv, colv])
                    t = plsc.load_gather(tbuf, [rowv, colv])
                    u = plsc.load_gather(ubuf, [rowv, colv])
                    acc = acc + (h + r) * u * t
                    nacc = nacc + t * t
                sbuf[pl.ds(_c * C + g * L, L)] = acc
                return nv + jnp.maximum(nacc - 1.0, 0.0)

            norm_vec = lax.fori_loop(0, C // L, group_body, norm_vec)

        pltpu.sync_copy(sbuf, score_out.at[pl.ds(base, BPW)])
        nbuf[...] = norm_vec
        pltpu.sync_copy(nbuf, norm_out.at[wid])

    return k(heads, rels, tails, user, ent_table, rel_table)


def kernel(heads, rels, tails, e1_embedded_user, ent_table, rel_table):
    B, D = e1_embedded_user.shape
    score, norm_parts = _sc_run(
        heads, rels, tails, e1_embedded_user, ent_table, rel_table, B, D)
    return score, jnp.sum(norm_parts)


# speedup vs baseline: 1.0110x; 1.0110x over previous
import functools

import jax
import jax.numpy as jnp
from jax import lax
from jax.experimental import pallas as pl
from jax.experimental.pallas import tpu as pltpu
from jax.experimental.pallas import tpu_sc as plsc

L = 16


@functools.partial(jax.jit, static_argnums=(6, 7))
def _sc_run(heads, rels, tails, user2, ent2, rel2, B, D):
    info = plsc.get_sparse_core_info()
    NC, NS = info.num_cores, info.num_subcores
    NW = NC * NS
    BPW = B // NW
    C = 128
    NCH = BPW // C
    W = 2 * D
    assert BPW * NW == B and NCH * C == BPW

    mesh = plsc.VectorSubcoreMesh(core_axis_name="c", subcore_axis_name="s")

    @functools.partial(
        pl.kernel,
        mesh=mesh,
        compiler_params=pltpu.CompilerParams(
            needs_layout_passes=False, use_tc_tiling_on_sc=True),
        out_type=(
            jax.ShapeDtypeStruct((B,), jnp.float32),
            jax.ShapeDtypeStruct((NW, L), jnp.float32),
        ),
        scratch_types=[
            pltpu.VMEM((C,), jnp.int32),
            pltpu.VMEM((C,), jnp.int32),
            pltpu.VMEM((C,), jnp.int32),
            pltpu.VMEM((C,), jnp.int32),
            pltpu.VMEM((C,), jnp.int32),
            pltpu.VMEM((C,), jnp.int32),
            pltpu.VMEM((C, W), jnp.float32),
            pltpu.VMEM((C, W), jnp.float32),
            pltpu.VMEM((C, W), jnp.float32),
            pltpu.VMEM((C // 2, W), jnp.float32),
            pltpu.VMEM((BPW,), jnp.float32),
            pltpu.VMEM((L,), jnp.float32),
            pltpu.SemaphoreType.DMA,
        ],
    )
    def k(heads_hbm, rels_hbm, tails_hbm, user_hbm, ent_hbm, rel_hbm,
          score_out, norm_out,
          hidx, ridx, tidx, hpix, rpix, tpix,
          hbuf, rbuf, tbuf, ubuf, sbuf, nbuf, sem):
        wid = lax.axis_index("s") * NC + lax.axis_index("c")
        base = pl.multiple_of(wid * BPW, BPW)
        lane = lax.iota(jnp.int32, L)
        lhalf = lax.shift_right_logical(lane, 1)
        lodd64 = lax.shift_left(lax.bitwise_and(lane, 1), 6)
        norm_vec = jnp.zeros((L,), jnp.float32)

        for c in range(NCH):
            r0 = base + c * C
            pltpu.sync_copy(heads_hbm.at[pl.ds(r0, C)], hidx)
            pltpu.sync_copy(rels_hbm.at[pl.ds(r0, C)], ridx)
            pltpu.sync_copy(tails_hbm.at[pl.ds(r0, C)], tidx)
            for i in range(C // L):
                s = pl.ds(i * L, L)
                hpix[s] = lax.shift_right_logical(hidx[s], 1)
                rpix[s] = lax.shift_right_logical(ridx[s], 1)
                tpix[s] = lax.shift_right_logical(tidx[s], 1)
            cp_h = pltpu.async_copy(ent_hbm.at[hpix], hbuf, sem)
            cp_r = pltpu.async_copy(rel_hbm.at[rpix], rbuf, sem)
            cp_t = pltpu.async_copy(ent_hbm.at[tpix], tbuf, sem)
            cp_u = pltpu.async_copy(
                user_hbm.at[pl.ds(pl.multiple_of(r0 // 2, C // 2), C // 2)],
                ubuf, sem)
            cp_h.wait()
            cp_r.wait()
            cp_t.wait()
            cp_u.wait()

            def group_body(g, nv, _c=c):
                rowv = lane + g * L
                urow = lhalf + g * (L // 2)
                gs = pl.ds(g * L, L)
                hcol = lax.shift_left(lax.bitwise_and(hidx[gs], 1), 6)
                rcol = lax.shift_left(lax.bitwise_and(ridx[gs], 1), 6)
                tcol = lax.shift_left(lax.bitwise_and(tidx[gs], 1), 6)
                acc = jnp.zeros((L,), jnp.float32)
                nacc = jnp.zeros((L,), jnp.float32)
                for d in range(D):
                    h = plsc.load_gather(hbuf, [rowv, hcol + d])
                    r = plsc.load_gather(rbuf, [rowv, rcol + d])
                    t = plsc.load_gather(tbuf, [rowv, tcol + d])
                    u = plsc.load_gather(ubuf, [urow, lodd64 + d])
                    acc = acc + (h + r) * u * t
                    nacc = nacc + t * t
                sbuf[pl.ds(_c * C + g * L, L)] = acc
                return nv + jnp.maximum(nacc - 1.0, 0.0)

            norm_vec = lax.fori_loop(0, C // L, group_body, norm_vec)

        pltpu.sync_copy(sbuf, score_out.at[pl.ds(base, BPW)])
        nbuf[...] = norm_vec
        pltpu.sync_copy(nbuf, norm_out.at[wid])

    return k(heads, rels, tails, user2, ent2, rel2)


def kernel(heads, rels, tails, e1_embedded_user, ent_table, rel_table):
    B, D = e1_embedded_user.shape
    ent2 = ent_table.reshape(ent_table.shape[0] // 2, 2 * D)
    rel2 = rel_table.reshape(rel_table.shape[0] // 2, 2 * D)
    user2 = e1_embedded_user.reshape(B // 2, 2 * D)
    score, norm_parts = _sc_run(heads, rels, tails, user2, ent2, rel2, B, D)
    return score, jnp.sum(norm_parts)
